# baseline (speedup 1.0000x reference)
import functools

import jax
import jax.numpy as jnp
from jax import lax
from jax.experimental import pallas as pl
from jax.experimental.pallas import tpu as pltpu
from jax.experimental.pallas import tpu_sc as plsc

_B = 4096
_D = 32
_K = 5
_EPS = 1e-8

_NC = 2
_NS = 16
_L = 16
_NW = _NC * _NS
_R = _B // _NW
_F = _R * _K
_NG = _F // _L


def _sc_neg_products(batch_users, negidx_flat, total_user, total_sub):
  mesh = plsc.VectorSubcoreMesh(core_axis_name="c", subcore_axis_name="s")

  @functools.partial(
      pl.kernel,
      mesh=mesh,
      compiler_params=pltpu.CompilerParams(use_tc_tiling_on_sc=False),
      out_type=jax.ShapeDtypeStruct((_B * _K,), jnp.float32),
      scratch_types=[
          pltpu.VMEM((_R,), jnp.int32),
          pltpu.VMEM((_F,), jnp.int32),
          pltpu.VMEM((_R, _D), jnp.float32),
          pltpu.VMEM((_F, _D), jnp.float32),
          pltpu.VMEM((_F,), jnp.float32),
          pltpu.SemaphoreType.DMA,
      ],
  )
  def k(bu_hbm, ni_hbm, tu_hbm, ts_hbm, out_hbm,
        idxu_v, idxn_v, urows_v, nrows_v, prod_v, sem):
    wid = lax.axis_index("s") * _NC + lax.axis_index("c")
    base = wid * _R
    pltpu.sync_copy(bu_hbm.at[pl.ds(base, _R)], idxu_v)
    pltpu.sync_copy(ni_hbm.at[pl.ds(wid * _F, _F)], idxn_v)
    copies = [pltpu.async_copy(tu_hbm.at[idxu_v], urows_v, sem)]
    for j in range(_K):
      copies.append(pltpu.async_copy(
          ts_hbm.at[idxn_v.at[pl.ds(j * _R, _R)]],
          nrows_v.at[pl.ds(j * _R, _R)], sem))
    for c in copies:
      c.wait()

    lanes = lax.iota(jnp.int32, _L)

    def merge(a, b, sh):
      ax = a + a.at[lanes ^ sh].get(mode="promise_in_bounds")
      bx = b + b.at[lanes ^ sh].get(mode="promise_in_bounds")
      return jnp.where((lanes & sh) == 0, ax, bx)

    def group(g, carry):
      gbase = g * _L
      v = []
      for j in range(_L):
        i = gbase + j
        ui = lax.shift_right_logical(i * 6554, 15)
        u0 = urows_v[ui, pl.ds(0, _L)]
        u1 = urows_v[ui, pl.ds(_L, _L)]
        n0 = nrows_v[i, pl.ds(0, _L)]
        n1 = nrows_v[i, pl.ds(_L, _L)]
        v.append(u0 * n0 + u1 * n1)
      c = [merge(v[r], v[r + 8], 8) for r in range(8)]
      d = [merge(c[r], c[r + 4], 4) for r in range(4)]
      e = [merge(d[r], d[r + 2], 2) for r in range(2)]
      prod_v[pl.ds(gbase, _L)] = merge(e[0], e[1], 1)
      return carry

    lax.fori_loop(0, _NG, group, 0)
    pltpu.sync_copy(prod_v, out_hbm.at[pl.ds(wid * _F, _F)])

  return k(batch_users, negidx_flat, total_user, total_sub)


def _tc_loss(ue, se, nprod2d):
  def body(ue_ref, se_ref, np_ref, out_ref):
    x = jnp.sum(ue_ref[...] * se_ref[...], axis=-1)
    pos = -jnp.log(1.0 / (1.0 + jnp.exp(-x)) + _EPS)
    y = np_ref[...]
    neg = jnp.log(1.0 / (1.0 + jnp.exp(y)) + _EPS)
    loss = jnp.sum(pos) / _B + jnp.sum(neg) / (_B * _K)
    out_ref[...] = jnp.reshape(loss, (1, 1))

  out = pl.pallas_call(
      body,
      out_shape=jax.ShapeDtypeStruct((1, 1), jnp.float32),
  )(ue, se, nprod2d)
  return out[0, 0]


def kernel(user_embeddings, subreddit_embeddings, batch_users, batch_subreddits,
           total_user_embeddings, total_subreddit_embeddings, negative_indices):
  del batch_subreddits
  negidx_flat = negative_indices.reshape(_B * _K)
  nprod = _sc_neg_products(batch_users, negidx_flat,
                           total_user_embeddings, total_subreddit_embeddings)
  return _tc_loss(user_embeddings, subreddit_embeddings,
                  nprod.reshape(_B * _K // 128, 128))


# speedup vs baseline: 1.5603x; 1.5603x over previous
import functools

import jax
import jax.numpy as jnp
from jax import lax
from jax.experimental import pallas as pl
from jax.experimental.pallas import tpu as pltpu
from jax.experimental.pallas import tpu_sc as plsc

_B = 4096
_D = 32
_K = 5
_EPS = 1e-8

_NC = 2
_NS = 16
_L = 16
_NW = _NC * _NS
_R = _B // _NW
_F = _R * _K
_NG = _F // _L


def _sc_neg_products(batch_users, negidx_flat, total_user, total_sub):
  mesh = plsc.VectorSubcoreMesh(core_axis_name="c", subcore_axis_name="s")

  @functools.partial(
      pl.kernel,
      mesh=mesh,
      out_type=jax.ShapeDtypeStruct((_B * _K,), jnp.float32),
      scratch_types=[
          pltpu.VMEM((_R,), jnp.int32),
          pltpu.VMEM((_F,), jnp.int32),
          pltpu.VMEM((_R, _D), jnp.float32),
          pltpu.VMEM((_F, _D), jnp.float32),
          pltpu.VMEM((_F,), jnp.float32),
          pltpu.SemaphoreType.DMA,
      ],
  )
  def k(bu_hbm, ni_hbm, tu_hbm, ts_hbm, out_hbm,
        idxu_v, idxn_v, urows_v, nrows_v, prod_v, sem):
    wid = lax.axis_index("s") * _NC + lax.axis_index("c")
    base = wid * _R
    pltpu.sync_copy(bu_hbm.at[pl.ds(base, _R)], idxu_v)
    pltpu.sync_copy(ni_hbm.at[pl.ds(wid * _F, _F)], idxn_v)

    def fetch_u(c, carry):
      iv = idxu_v[pl.ds(c * _L, _L)]
      for j in range(_L):
        pltpu.async_copy(tu_hbm.at[pl.ds(iv[j], 1)],
                         urows_v.at[pl.ds(c * _L + j, 1)], sem)
      return carry

    def fetch_n(c, carry):
      iv = idxn_v[pl.ds(c * _L, _L)]
      for j in range(_L):
        pltpu.async_copy(ts_hbm.at[pl.ds(iv[j], 1)],
                         nrows_v.at[pl.ds(c * _L + j, 1)], sem)
      return carry

    lax.fori_loop(0, _R // _L, fetch_u, 0)
    lax.fori_loop(0, _NG, fetch_n, 0)
    pltpu.make_async_copy(tu_hbm.at[pl.ds(0, _R)], urows_v, sem).wait()
    pltpu.make_async_copy(ts_hbm.at[pl.ds(0, _F)], nrows_v, sem).wait()

    lanes = lax.iota(jnp.int32, _L)

    def merge(a, b, sh):
      ax = a + a.at[lanes ^ sh].get(mode="promise_in_bounds")
      bx = b + b.at[lanes ^ sh].get(mode="promise_in_bounds")
      return jnp.where((lanes & sh) == 0, ax, bx)

    def group(g, carry):
      gbase = g * _L
      v = []
      for j in range(_L):
        i = gbase + j
        ui = lax.shift_right_logical(i * 6554, 15)
        u0 = urows_v[ui, pl.ds(0, _L)]
        u1 = urows_v[ui, pl.ds(_L, _L)]
        n0 = nrows_v[i, pl.ds(0, _L)]
        n1 = nrows_v[i, pl.ds(_L, _L)]
        v.append(u0 * n0 + u1 * n1)
      c = [merge(v[r], v[r + 8], 8) for r in range(8)]
      d = [merge(c[r], c[r + 4], 4) for r in range(4)]
      e = [merge(d[r], d[r + 2], 2) for r in range(2)]
      prod_v[pl.ds(gbase, _L)] = merge(e[0], e[1], 1)
      return carry

    lax.fori_loop(0, _NG, group, 0)
    pltpu.sync_copy(prod_v, out_hbm.at[pl.ds(wid * _F, _F)])

  return k(batch_users, negidx_flat, total_user, total_sub)


def _tc_loss(ue, se, nprod2d):
  def body(ue_ref, se_ref, np_ref, out_ref):
    x = jnp.sum(ue_ref[...] * se_ref[...], axis=-1)
    pos = -jnp.log(1.0 / (1.0 + jnp.exp(-x)) + _EPS)
    y = np_ref[...]
    neg = jnp.log(1.0 / (1.0 + jnp.exp(y)) + _EPS)
    loss = jnp.sum(pos) / _B + jnp.sum(neg) / (_B * _K)
    out_ref[...] = jnp.reshape(loss, (1, 1))

  out = pl.pallas_call(
      body,
      out_shape=jax.ShapeDtypeStruct((1, 1), jnp.float32),
  )(ue, se, nprod2d)
  return out[0, 0]


def kernel(user_embeddings, subreddit_embeddings, batch_users, batch_subreddits,
           total_user_embeddings, total_subreddit_embeddings, negative_indices):
  del batch_subreddits
  negidx_flat = negative_indices.reshape(_B * _K)
  nprod = _sc_neg_products(batch_users, negidx_flat,
                           total_user_embeddings, total_subreddit_embeddings)
  return _tc_loss(user_embeddings, subreddit_embeddings,
                  nprod.reshape(_B * _K // 128, 128))


# speedup vs baseline: 1.6279x; 1.0433x over previous
import functools

import jax
import jax.numpy as jnp
from jax import lax
from jax.experimental import pallas as pl
from jax.experimental.pallas import tpu as pltpu
from jax.experimental.pallas import tpu_sc as plsc

_B = 4096
_D = 32
_K = 5
_EPS = 1e-8

_NC = 2
_NS = 16
_L = 16
_NW = _NC * _NS
_R = _B // _NW
_F = _R * _K

_W = 5120


def _pack(table_t, n_rows):
  n = table_t.shape[1]
  grid = (n + _W - 1) // _W

  def body(in_ref, out_ref):
    x = in_ref[...]
    eye = jnp.eye(_D, dtype=jnp.float32)
    y = jax.lax.dot_general(x, eye, (((0,), (0,)), ((), ())),
                            preferred_element_type=jnp.float32)
    q = _W // 4
    out_ref[...] = jnp.concatenate(
        [y[0:q], y[q:2 * q], y[2 * q:3 * q], y[3 * q:4 * q]], axis=1)

  return pl.pallas_call(
      body,
      grid=(grid,),
      in_specs=[pl.BlockSpec((_D, _W), lambda i: (0, i))],
      out_specs=pl.BlockSpec((_W // 4, 128), lambda i: (i, 0)),
      out_shape=jax.ShapeDtypeStruct((grid * (_W // 4), 128), jnp.float32),
  )(table_t)


def _sc_neg_products(batch_users, negidx_flat, tu_p, ts_p):
  mesh = plsc.VectorSubcoreMesh(core_axis_name="c", subcore_axis_name="s")

  @functools.partial(
      pl.kernel,
      mesh=mesh,
      out_type=jax.ShapeDtypeStruct((_B * _K,), jnp.float32),
      scratch_types=[
          pltpu.VMEM((_R,), jnp.int32),
          pltpu.VMEM((_F,), jnp.int32),
          pltpu.VMEM((_R, 128), jnp.float32),
          pltpu.VMEM((_R, _D), jnp.float32),
          pltpu.VMEM((_F, 128), jnp.float32),
          pltpu.VMEM((_F,), jnp.float32),
          pltpu.SemaphoreType.DMA,
      ],
  )
  def k(bu_hbm, ni_hbm, tu_hbm, ts_hbm, out_hbm,
        idxu_v, idxn_v, urows_v, ucomp_v, nrows_v, prod_v, sem):
    wid = lax.axis_index("s") * _NC + lax.axis_index("c")
    base = wid * _R
    pltpu.sync_copy(bu_hbm.at[pl.ds(base, _R)], idxu_v)
    pltpu.sync_copy(ni_hbm.at[pl.ds(wid * _F, _F)], idxn_v)

    def prow(i):
      q = lax.shift_right_logical(
          lax.shift_right_logical(i, 10) * 6554, 15)
      v = i - q * 5120
      t = lax.shift_right_logical(
          lax.shift_right_logical(v, 8) * 6554, 15)
      return q * 1280 + (v - t * 1280)

    def fetch_u(c, carry):
      iv = idxu_v[pl.ds(c * _L, _L)]
      for j in range(_L):
        pltpu.async_copy(tu_hbm.at[pl.ds(prow(iv[j]), 1)],
                         urows_v.at[pl.ds(c * _L + j, 1)], sem)
      return carry

    def fetch_n(c, carry):
      iv = idxn_v[pl.ds(c * _L, _L)]
      for j in range(_L):
        pltpu.async_copy(ts_hbm.at[pl.ds(prow(iv[j]), 1)],
                         nrows_v.at[pl.ds(c * _L + j, 1)], sem)
      return carry

    lax.fori_loop(0, _R // _L, fetch_u, 0)
    lax.fori_loop(0, _F // _L, fetch_n, 0)
    pltpu.make_async_copy(tu_hbm.at[pl.ds(0, _R)], urows_v, sem).wait()
    pltpu.make_async_copy(ts_hbm.at[pl.ds(0, _F)], nrows_v, sem).wait()

    def slot(i):
      q = lax.shift_right_logical(
          lax.shift_right_logical(i, 10) * 6554, 15)
      v = i - q * 5120
      return lax.shift_right_logical(
          lax.shift_right_logical(v, 8) * 6554, 15)

    def pick(ref, r, t):
      ps = [ref[r, pl.ds(o * _L, _L)] for o in range(8)]
      h0 = jnp.where(t == 0, ps[0], jnp.where(t == 1, ps[2],
                     jnp.where(t == 2, ps[4], ps[6])))
      h1 = jnp.where(t == 0, ps[1], jnp.where(t == 1, ps[3],
                     jnp.where(t == 2, ps[5], ps[7])))
      return h0, h1

    def extract_u(c, carry):
      iv = idxu_v[pl.ds(c * _L, _L)]
      for j in range(_L):
        r = c * _L + j
        h0, h1 = pick(urows_v, r, slot(iv[j]))
        ucomp_v[r, pl.ds(0, _L)] = h0
        ucomp_v[r, pl.ds(_L, _L)] = h1
      return carry

    lax.fori_loop(0, _R // _L, extract_u, 0)

    lanes = lax.iota(jnp.int32, _L)

    def merge(a, b, sh):
      ax = a + a.at[lanes ^ sh].get(mode="promise_in_bounds")
      bx = b + b.at[lanes ^ sh].get(mode="promise_in_bounds")
      return jnp.where((lanes & sh) == 0, ax, bx)

    def group(g, carry):
      gbase = g * _L
      ivn = idxn_v[pl.ds(gbase, _L)]
      v = []
      for j in range(_L):
        i = gbase + j
        ui = lax.shift_right_logical(i * 6554, 15)
        u0 = ucomp_v[ui, pl.ds(0, _L)]
        u1 = ucomp_v[ui, pl.ds(_L, _L)]
        n0, n1 = pick(nrows_v, i, slot(ivn[j]))
        v.append(u0 * n0 + u1 * n1)
      c = [merge(v[r], v[r + 8], 8) for r in range(8)]
      d = [merge(c[r], c[r + 4], 4) for r in range(4)]
      e = [merge(d[r], d[r + 2], 2) for r in range(2)]
      prod_v[pl.ds(gbase, _L)] = merge(e[0], e[1], 1)
      return carry

    lax.fori_loop(0, _F // _L, group, 0)
    pltpu.sync_copy(prod_v, out_hbm.at[pl.ds(wid * _F, _F)])

  return k(batch_users, negidx_flat, tu_p, ts_p)


def _tc_loss(ue, se, nprod2d):
  def body(ue_ref, se_ref, np_ref, out_ref):
    x = jnp.sum(ue_ref[...] * se_ref[...], axis=-1)
    pos = -jnp.log(1.0 / (1.0 + jnp.exp(-x)) + _EPS)
    y = np_ref[...]
    neg = jnp.log(1.0 / (1.0 + jnp.exp(y)) + _EPS)
    loss = jnp.sum(pos) / _B + jnp.sum(neg) / (_B * _K)
    out_ref[...] = jnp.reshape(loss, (1, 1))

  out = pl.pallas_call(
      body,
      out_shape=jax.ShapeDtypeStruct((1, 1), jnp.float32),
  )(ue, se, nprod2d)
  return out[0, 0]


def kernel(user_embeddings, subreddit_embeddings, batch_users, batch_subreddits,
           total_user_embeddings, total_subreddit_embeddings, negative_indices):
  del batch_subreddits
  negidx_flat = negative_indices.reshape(_B * _K)
  tu_p = _pack(total_user_embeddings.T, 100000)
  ts_p = _pack(total_subreddit_embeddings.T, 1000000)
  nprod = _sc_neg_products(batch_users, negidx_flat, tu_p, ts_p)
  return _tc_loss(user_embeddings, subreddit_embeddings,
                  nprod.reshape(_B * _K // 128, 128))


# speedup vs baseline: 2.3876x; 1.4667x over previous
import functools

import jax
import jax.numpy as jnp
from jax import lax
from jax.experimental import pallas as pl
from jax.experimental.pallas import tpu as pltpu
from jax.experimental.pallas import tpu_sc as plsc

_B = 4096
_D = 32
_K = 5
_EPS = 1e-8

_NC = 2
_NS = 16
_L = 16
_NW = _NC * _NS
_R = _B // _NW
_F = _R * _K

_W = 10240


def _pack(table_t, n_rows):
  n = table_t.shape[1]
  grid = (n + _W - 1) // _W

  def body(in_ref, out_ref):
    x = in_ref[...]
    q = _W // 4
    acc = None
    for t in range(4):
      et = jnp.eye(_D, 128, k=_D * t, dtype=jnp.float32)
      yt = jax.lax.dot_general(x[:, t * q:(t + 1) * q], et,
                               (((0,), (0,)), ((), ())),
                               preferred_element_type=jnp.float32)
      acc = yt if acc is None else acc + yt
    out_ref[...] = acc

  return pl.pallas_call(
      body,
      grid=(grid,),
      in_specs=[pl.BlockSpec((_D, _W), lambda i: (0, i))],
      out_specs=pl.BlockSpec((_W // 4, 128), lambda i: (i, 0)),
      out_shape=jax.ShapeDtypeStruct((grid * (_W // 4), 128), jnp.float32),
  )(table_t)


def _sc_neg_products(batch_users, negidx_flat, tu_p, ts_p):
  mesh = plsc.VectorSubcoreMesh(core_axis_name="c", subcore_axis_name="s")

  @functools.partial(
      pl.kernel,
      mesh=mesh,
      out_type=jax.ShapeDtypeStruct((_B * _K,), jnp.float32),
      scratch_types=[
          pltpu.VMEM((_R,), jnp.int32),
          pltpu.VMEM((_F,), jnp.int32),
          pltpu.VMEM((_R, 128), jnp.float32),
          pltpu.VMEM((_R, _D), jnp.float32),
          pltpu.VMEM((_F, 128), jnp.float32),
          pltpu.VMEM((_F,), jnp.float32),
          pltpu.SemaphoreType.DMA,
      ],
  )
  def k(bu_hbm, ni_hbm, tu_hbm, ts_hbm, out_hbm,
        idxu_v, idxn_v, urows_v, ucomp_v, nrows_v, prod_v, sem):
    wid = lax.axis_index("s") * _NC + lax.axis_index("c")
    base = wid * _R
    pltpu.sync_copy(bu_hbm.at[pl.ds(base, _R)], idxu_v)
    pltpu.sync_copy(ni_hbm.at[pl.ds(wid * _F, _F)], idxn_v)

    def prow(i):
      q = lax.shift_right_logical(
          lax.shift_right_logical(i, 11) * 6554, 15)
      v = i - q * 10240
      t = lax.shift_right_logical(
          lax.shift_right_logical(v, 9) * 6554, 15)
      return q * 2560 + (v - t * 2560)

    def fetch_u(c, carry):
      iv = idxu_v[pl.ds(c * _L, _L)]
      for j in range(_L):
        pltpu.async_copy(tu_hbm.at[pl.ds(prow(iv[j]), 1)],
                         urows_v.at[pl.ds(c * _L + j, 1)], sem)
      return carry

    def fetch_n(c, carry):
      iv = idxn_v[pl.ds(c * _L, _L)]
      for j in range(_L):
        pltpu.async_copy(ts_hbm.at[pl.ds(prow(iv[j]), 1)],
                         nrows_v.at[pl.ds(c * _L + j, 1)], sem)
      return carry

    lax.fori_loop(0, _R // _L, fetch_u, 0)
    lax.fori_loop(0, _F // _L, fetch_n, 0)
    pltpu.make_async_copy(tu_hbm.at[pl.ds(0, _R)], urows_v, sem).wait()
    pltpu.make_async_copy(ts_hbm.at[pl.ds(0, _F)], nrows_v, sem).wait()

    def slot(i):
      q = lax.shift_right_logical(
          lax.shift_right_logical(i, 11) * 6554, 15)
      v = i - q * 10240
      return lax.shift_right_logical(
          lax.shift_right_logical(v, 9) * 6554, 15)

    def pick(ref, r, t):
      ps = [ref[r, pl.ds(o * _L, _L)] for o in range(8)]
      h0 = jnp.where(t == 0, ps[0], jnp.where(t == 1, ps[2],
                     jnp.where(t == 2, ps[4], ps[6])))
      h1 = jnp.where(t == 0, ps[1], jnp.where(t == 1, ps[3],
                     jnp.where(t == 2, ps[5], ps[7])))
      return h0, h1

    def extract_u(c, carry):
      iv = idxu_v[pl.ds(c * _L, _L)]
      for j in range(_L):
        r = c * _L + j
        h0, h1 = pick(urows_v, r, slot(iv[j]))
        ucomp_v[r, pl.ds(0, _L)] = h0
        ucomp_v[r, pl.ds(_L, _L)] = h1
      return carry

    lax.fori_loop(0, _R // _L, extract_u, 0)

    lanes = lax.iota(jnp.int32, _L)

    def merge(a, b, sh):
      ax = a + a.at[lanes ^ sh].get(mode="promise_in_bounds")
      bx = b + b.at[lanes ^ sh].get(mode="promise_in_bounds")
      return jnp.where((lanes & sh) == 0, ax, bx)

    def group(g, carry):
      gbase = g * _L
      ivn = idxn_v[pl.ds(gbase, _L)]
      v = []
      for j in range(_L):
        i = gbase + j
        ui = lax.shift_right_logical(i * 6554, 15)
        u0 = ucomp_v[ui, pl.ds(0, _L)]
        u1 = ucomp_v[ui, pl.ds(_L, _L)]
        n0, n1 = pick(nrows_v, i, slot(ivn[j]))
        v.append(u0 * n0 + u1 * n1)
      c = [merge(v[r], v[r + 8], 8) for r in range(8)]
      d = [merge(c[r], c[r + 4], 4) for r in range(4)]
      e = [merge(d[r], d[r + 2], 2) for r in range(2)]
      prod_v[pl.ds(gbase, _L)] = merge(e[0], e[1], 1)
      return carry

    lax.fori_loop(0, _F // _L, group, 0)
    pltpu.sync_copy(prod_v, out_hbm.at[pl.ds(wid * _F, _F)])

  return k(batch_users, negidx_flat, tu_p, ts_p)


def _tc_loss(ue, se, nprod2d):
  def body(ue_ref, se_ref, np_ref, out_ref):
    x = jnp.sum(ue_ref[...] * se_ref[...], axis=-1)
    pos = -jnp.log(1.0 / (1.0 + jnp.exp(-x)) + _EPS)
    y = np_ref[...]
    neg = jnp.log(1.0 / (1.0 + jnp.exp(y)) + _EPS)
    loss = jnp.sum(pos) / _B + jnp.sum(neg) / (_B * _K)
    out_ref[...] = jnp.reshape(loss, (1, 1))

  out = pl.pallas_call(
      body,
      out_shape=jax.ShapeDtypeStruct((1, 1), jnp.float32),
  )(ue, se, nprod2d)
  return out[0, 0]


def kernel(user_embeddings, subreddit_embeddings, batch_users, batch_subreddits,
           total_user_embeddings, total_subreddit_embeddings, negative_indices):
  del batch_subreddits
  negidx_flat = negative_indices.reshape(_B * _K)
  tu_p = _pack(total_user_embeddings.T, 100000)
  ts_p = _pack(total_subreddit_embeddings.T, 1000000)
  nprod = _sc_neg_products(batch_users, negidx_flat, tu_p, ts_p)
  return _tc_loss(user_embeddings, subreddit_embeddings,
                  nprod.reshape(_B * _K // 128, 128))
